# baseline (speedup 1.0000x reference)
import math

import jax
import jax.numpy as jnp
from jax.experimental import pallas as pl
from jax.experimental.pallas import tpu as pltpu

S = 512
U = 1024
H = 12
C = 10
NBK = 32
NCLS = 4


def _pe_body(a_ref, bj_ref, im_ref, w1r_ref, o_ref, t_ref):
    b = pl.program_id(1)

    @pl.when(b == 0)
    def _build_t():
        t = jax.lax.broadcasted_iota(jnp.int32, (NBK, U), 1)
        r = jax.lax.broadcasted_iota(jnp.int32, (NBK, U), 0)
        d = jnp.where(t < S, t, t - U)
        n = -d
        ret = jnp.where(n < 0, NBK // 2, 0)
        na = jnp.abs(n)
        max_exact = NBK // 4
        nf = jnp.maximum(na, 1).astype(jnp.float32)
        vl = max_exact + (
            jnp.log(nf / max_exact)
            / math.log(128.0 / max_exact)
            * (NBK // 2 - max_exact)
        ).astype(jnp.int32)
        vl = jnp.minimum(vl, NBK // 2 - 1)
        f = ret + jnp.where(na < max_exact, na, vl)
        oh = (f == r).astype(jnp.float32)
        d2 = jnp.dot(
            w1r_ref[0], oh, preferred_element_type=jnp.float32
        ).astype(jnp.bfloat16)
        for c in range(C):
            row = jnp.broadcast_to(d2[c : c + 1, :], (S, U))
            t_ref[c] = pltpu.roll(row, 0, 1, stride=1, stride_axis=0)[:, :S]

    im = im_ref[0]
    brow = bj_ref[0, 0, :]
    cj = [(brow == j)[None, :] for j in range(1, NCLS)]
    ri = [im[:, i : i + 1] > 0.5 for i in range(1, NCLS)]

    t0 = t_ref[0]
    w = []
    for i in range(1, NCLS):
        wi = jnp.where(
            cj[0],
            t_ref[(i - 1) * 3 + 1],
            jnp.where(
                cj[1],
                t_ref[(i - 1) * 3 + 2],
                jnp.where(cj[2], t_ref[(i - 1) * 3 + 3], t0),
            ),
        )
        w.append(wi)
    v = jnp.where(ri[0], w[0], jnp.where(ri[1], w[1], jnp.where(ri[2], w[2], t0)))
    o_ref[0, 0] = a_ref[0, 0] + v.astype(jnp.float32)


def kernel(attention_scores, b_seq, W1):
    B = attention_scores.shape[0]
    w1r = jnp.transpose(W1, (2, 1, 0))
    bj = b_seq.reshape(B, 1, S)
    im = (b_seq[:, :, None] == jnp.arange(NCLS, dtype=b_seq.dtype)).astype(
        jnp.float32
    )
    out = pl.pallas_call(
        _pe_body,
        grid=(H, B),
        in_specs=[
            pl.BlockSpec((1, 1, S, S), lambda h, b: (b, h, 0, 0)),
            pl.BlockSpec((1, 1, S), lambda h, b: (b, 0, 0)),
            pl.BlockSpec((1, S, NCLS), lambda h, b: (b, 0, 0)),
            pl.BlockSpec((1, C, NBK), lambda h, b: (h, 0, 0)),
        ],
        out_specs=pl.BlockSpec((1, 1, S, S), lambda h, b: (b, h, 0, 0)),
        out_shape=jax.ShapeDtypeStruct(attention_scores.shape, jnp.float32),
        scratch_shapes=[pltpu.VMEM((C, S, S), jnp.bfloat16)],
        compiler_params=pltpu.CompilerParams(
            dimension_semantics=("parallel", "arbitrary")
        ),
    )(attention_scores, bj, im, w1r)
    return out


# speedup vs baseline: 1.0327x; 1.0327x over previous
import math

import jax
import jax.numpy as jnp
from jax.experimental import pallas as pl
from jax.experimental.pallas import tpu as pltpu

S = 512
U = 1024
H = 12
C = 10
NBK = 32
NCLS = 4


def _pe_body(a_ref, bj_ref, im_ref, w1r_ref, o_ref, t_ref, d2_ref):
    h = pl.program_id(0)
    b = pl.program_id(1)

    @pl.when(jnp.logical_and(h == 0, b == 0))
    def _build_d2():
        t = jax.lax.broadcasted_iota(jnp.int32, (NBK, U), 1)
        r = jax.lax.broadcasted_iota(jnp.int32, (NBK, U), 0)
        d = jnp.where(t < S, t, t - U)
        n = -d
        ret = jnp.where(n < 0, NBK // 2, 0)
        na = jnp.abs(n)
        max_exact = NBK // 4
        nf = jnp.maximum(na, 1).astype(jnp.float32)
        vl = max_exact + (
            jnp.log(nf / max_exact)
            / math.log(128.0 / max_exact)
            * (NBK // 2 - max_exact)
        ).astype(jnp.int32)
        vl = jnp.minimum(vl, NBK // 2 - 1)
        f = ret + jnp.where(na < max_exact, na, vl)
        oh = (f == r).astype(jnp.float32)
        w1f = w1r_ref[...].reshape(H * C, NBK)
        d2_ref[...] = jnp.dot(
            w1f, oh, preferred_element_type=jnp.float32
        ).astype(jnp.bfloat16).reshape(H, C, U)

    @pl.when(b == 0)
    def _build_t():
        d2 = d2_ref[h]
        for c in range(C):
            row = jnp.broadcast_to(d2[c : c + 1, :], (S, U))
            t_ref[c] = pltpu.roll(row, 0, 1, stride=1, stride_axis=0)[:, :S]

    im = im_ref[0]
    brow = bj_ref[0, 0, :]
    cj = [(brow == j)[None, :] for j in range(1, NCLS)]
    ri = [im[:, i : i + 1] > 0.5 for i in range(1, NCLS)]

    t0 = t_ref[0]
    w = []
    for i in range(1, NCLS):
        wi = jnp.where(
            cj[0],
            t_ref[(i - 1) * 3 + 1],
            jnp.where(
                cj[1],
                t_ref[(i - 1) * 3 + 2],
                jnp.where(cj[2], t_ref[(i - 1) * 3 + 3], t0),
            ),
        )
        w.append(wi)
    v = jnp.where(ri[0], w[0], jnp.where(ri[1], w[1], jnp.where(ri[2], w[2], t0)))
    o_ref[0, 0] = a_ref[0, 0] + v.astype(jnp.float32)


def kernel(attention_scores, b_seq, W1):
    B = attention_scores.shape[0]
    w1r = jnp.transpose(W1, (2, 1, 0))
    bj = b_seq.reshape(B, 1, S)
    im = (b_seq[:, :, None] == jnp.arange(NCLS, dtype=b_seq.dtype)).astype(
        jnp.float32
    )
    out = pl.pallas_call(
        _pe_body,
        grid=(H, B),
        in_specs=[
            pl.BlockSpec((1, 1, S, S), lambda h, b: (b, h, 0, 0)),
            pl.BlockSpec((1, 1, S), lambda h, b: (b, 0, 0)),
            pl.BlockSpec((1, S, NCLS), lambda h, b: (b, 0, 0)),
            pl.BlockSpec((H, C, NBK), lambda h, b: (0, 0, 0)),
        ],
        out_specs=pl.BlockSpec((1, 1, S, S), lambda h, b: (b, h, 0, 0)),
        out_shape=jax.ShapeDtypeStruct(attention_scores.shape, jnp.float32),
        scratch_shapes=[
            pltpu.VMEM((C, S, S), jnp.bfloat16),
            pltpu.VMEM((H, C, U), jnp.bfloat16),
        ],
        compiler_params=pltpu.CompilerParams(
            dimension_semantics=("arbitrary", "arbitrary")
        ),
    )(attention_scores, bj, im, w1r)
    return out


# speedup vs baseline: 1.2621x; 1.2222x over previous
import math

import jax
import jax.numpy as jnp
from jax.experimental import pallas as pl
from jax.experimental.pallas import tpu as pltpu

S = 512
U = 1024
H = 12
C = 10
NBK = 32
NCLS = 4
HP = 2


def _pe_body(a_ref, bj_ref, im_ref, w1r_ref, o_ref, t_ref, d2_ref):
    h = pl.program_id(0)
    b = pl.program_id(1)

    @pl.when(jnp.logical_and(h == 0, b == 0))
    def _build_d2():
        t = jax.lax.broadcasted_iota(jnp.int32, (NBK, U), 1)
        r = jax.lax.broadcasted_iota(jnp.int32, (NBK, U), 0)
        d = jnp.where(t < S, t, t - U)
        n = -d
        ret = jnp.where(n < 0, NBK // 2, 0)
        na = jnp.abs(n)
        max_exact = NBK // 4
        nf = jnp.maximum(na, 1).astype(jnp.float32)
        vl = max_exact + (
            jnp.log(nf / max_exact)
            / math.log(128.0 / max_exact)
            * (NBK // 2 - max_exact)
        ).astype(jnp.int32)
        vl = jnp.minimum(vl, NBK // 2 - 1)
        f = ret + jnp.where(na < max_exact, na, vl)
        oh = (f == r).astype(jnp.float32)
        w1f = w1r_ref[...].reshape(H * C, NBK)
        d2_ref[...] = jnp.dot(
            w1f, oh, preferred_element_type=jnp.float32
        ).astype(jnp.bfloat16).reshape(H, C, U)

    @pl.when(b == 0)
    def _build_t():
        for k in range(HP):
            d2 = d2_ref[h * HP + k]
            for c in range(C):
                row = jnp.broadcast_to(d2[c : c + 1, :], (S, U))
                t_ref[k, c] = pltpu.roll(row, 0, 1, stride=1, stride_axis=0)[
                    :, :S
                ]

    im = im_ref[0]
    brow = bj_ref[0, 0, :]
    cj = [(brow == j)[None, :] for j in range(1, NCLS)]
    ri = [im[:, i : i + 1] > 0.5 for i in range(1, NCLS)]

    for k in range(HP):
        t0 = t_ref[k, 0]
        w = []
        for i in range(1, NCLS):
            wi = jnp.where(
                cj[0],
                t_ref[k, (i - 1) * 3 + 1],
                jnp.where(
                    cj[1],
                    t_ref[k, (i - 1) * 3 + 2],
                    jnp.where(cj[2], t_ref[k, (i - 1) * 3 + 3], t0),
                ),
            )
            w.append(wi)
        v = jnp.where(
            ri[0], w[0], jnp.where(ri[1], w[1], jnp.where(ri[2], w[2], t0))
        )
        o_ref[0, k] = a_ref[0, k] + v.astype(jnp.float32)


def kernel(attention_scores, b_seq, W1):
    B = attention_scores.shape[0]
    w1r = jnp.transpose(W1, (2, 1, 0))
    bj = b_seq.reshape(B, 1, S)
    im = (b_seq[:, :, None] == jnp.arange(NCLS, dtype=b_seq.dtype)).astype(
        jnp.float32
    )
    out = pl.pallas_call(
        _pe_body,
        grid=(H // HP, B),
        in_specs=[
            pl.BlockSpec((1, HP, S, S), lambda h, b: (b, h, 0, 0)),
            pl.BlockSpec((1, 1, S), lambda h, b: (b, 0, 0)),
            pl.BlockSpec((1, S, NCLS), lambda h, b: (b, 0, 0)),
            pl.BlockSpec((H, C, NBK), lambda h, b: (0, 0, 0)),
        ],
        out_specs=pl.BlockSpec((1, HP, S, S), lambda h, b: (b, h, 0, 0)),
        out_shape=jax.ShapeDtypeStruct(attention_scores.shape, jnp.float32),
        scratch_shapes=[
            pltpu.VMEM((HP, C, S, S), jnp.bfloat16),
            pltpu.VMEM((H, C, U), jnp.bfloat16),
        ],
        compiler_params=pltpu.CompilerParams(
            dimension_semantics=("arbitrary", "arbitrary")
        ),
    )(attention_scores, bj, im, w1r)
    return out


# speedup vs baseline: 1.3697x; 1.0852x over previous
import math

import jax
import jax.numpy as jnp
from jax.experimental import pallas as pl
from jax.experimental.pallas import tpu as pltpu

S = 512
U = 1024
H = 12
C = 10
NBK = 32
NCLS = 4
HP = 4


def _pe_body(a_ref, bj_ref, im_ref, w1r_ref, o_ref, t_ref, d2_ref):
    h = pl.program_id(0)
    b = pl.program_id(1)

    @pl.when(jnp.logical_and(h == 0, b == 0))
    def _build_d2():
        t = jax.lax.broadcasted_iota(jnp.int32, (NBK, U), 1)
        r = jax.lax.broadcasted_iota(jnp.int32, (NBK, U), 0)
        d = jnp.where(t < S, t, t - U)
        n = -d
        ret = jnp.where(n < 0, NBK // 2, 0)
        na = jnp.abs(n)
        max_exact = NBK // 4
        nf = jnp.maximum(na, 1).astype(jnp.float32)
        vl = max_exact + (
            jnp.log(nf / max_exact)
            / math.log(128.0 / max_exact)
            * (NBK // 2 - max_exact)
        ).astype(jnp.int32)
        vl = jnp.minimum(vl, NBK // 2 - 1)
        f = ret + jnp.where(na < max_exact, na, vl)
        oh = (f == r).astype(jnp.float32)
        w1f = w1r_ref[...].reshape(H * C, NBK)
        d2_ref[...] = jnp.dot(
            w1f, oh, preferred_element_type=jnp.float32
        ).astype(jnp.bfloat16).reshape(H, C, U)

    @pl.when(b == 0)
    def _build_t():
        for k in range(HP):
            d2 = d2_ref[h * HP + k]
            for c in range(C):
                row = jnp.broadcast_to(d2[c : c + 1, :], (S, U))
                t_ref[k, c] = pltpu.roll(row, 0, 1, stride=1, stride_axis=0)[
                    :, :S
                ]

    im = im_ref[0]
    brow = bj_ref[0, 0, :]
    cj = [(brow == j)[None, :] for j in range(1, NCLS)]
    ri = [im[:, i : i + 1] > 0.5 for i in range(1, NCLS)]

    for k in range(HP):
        t0 = t_ref[k, 0]
        w = []
        for i in range(1, NCLS):
            wi = jnp.where(
                cj[0],
                t_ref[k, (i - 1) * 3 + 1],
                jnp.where(
                    cj[1],
                    t_ref[k, (i - 1) * 3 + 2],
                    jnp.where(cj[2], t_ref[k, (i - 1) * 3 + 3], t0),
                ),
            )
            w.append(wi)
        v = jnp.where(
            ri[0], w[0], jnp.where(ri[1], w[1], jnp.where(ri[2], w[2], t0))
        )
        o_ref[0, k] = a_ref[0, k] + v.astype(jnp.float32)


def kernel(attention_scores, b_seq, W1):
    B = attention_scores.shape[0]
    w1r = jnp.transpose(W1, (2, 1, 0))
    bj = b_seq.reshape(B, 1, S)
    im = (b_seq[:, :, None] == jnp.arange(NCLS, dtype=b_seq.dtype)).astype(
        jnp.float32
    )
    out = pl.pallas_call(
        _pe_body,
        grid=(H // HP, B),
        in_specs=[
            pl.BlockSpec((1, HP, S, S), lambda h, b: (b, h, 0, 0)),
            pl.BlockSpec((1, 1, S), lambda h, b: (b, 0, 0)),
            pl.BlockSpec((1, S, NCLS), lambda h, b: (b, 0, 0)),
            pl.BlockSpec((H, C, NBK), lambda h, b: (0, 0, 0)),
        ],
        out_specs=pl.BlockSpec((1, HP, S, S), lambda h, b: (b, h, 0, 0)),
        out_shape=jax.ShapeDtypeStruct(attention_scores.shape, jnp.float32),
        scratch_shapes=[
            pltpu.VMEM((HP, C, S, S), jnp.bfloat16),
            pltpu.VMEM((H, C, U), jnp.bfloat16),
        ],
        compiler_params=pltpu.CompilerParams(
            dimension_semantics=("arbitrary", "arbitrary")
        ),
    )(attention_scores, bj, im, w1r)
    return out
